# Initial kernel scaffold; baseline (speedup 1.0000x reference)
import jax
import jax.numpy as jnp
from jax.experimental import pallas as pl


def kernel(x, edge_index, W1, b1, W2, b2):
    raise NotImplementedError("write your pallas kernel here")


# speedup vs baseline: 35.9079x; 35.9079x over previous
import functools

import jax
import jax.numpy as jnp
from jax import lax
from jax.experimental import pallas as pl
from jax.experimental.pallas import tpu as pltpu
from jax.experimental.pallas import tpu_sc as plsc

N = 10000
NPAD = 10240
E = 320000
BLK = 128
NC, NS = 2, 16
NW = NC * NS
EPAD = 327680
ROWS = EPAD // BLK
RPW = ROWS // NW
NPT = NPAD // NS

def _sc_deg_c0_body(dstp, srcp, ones_h, zer_h, out_deg, out_c0,
                    dst_v, src_v, ones_v, upd_v, deg_sh, c0_sh):
    c = lax.axis_index("c")
    s = lax.axis_index("s")
    w = c * NS + s
    pltpu.sync_copy(zer_h, deg_sh.at[pl.ds(s * NPT, NPT)])
    pltpu.sync_copy(zer_h, c0_sh.at[pl.ds(s * NPT, NPT)])
    pltpu.sync_copy(ones_h, ones_v)
    pltpu.sync_copy(dstp.at[pl.ds(w * RPW, RPW)], dst_v)
    pltpu.sync_copy(srcp.at[pl.ds(w * RPW, RPW)], src_v)
    plsc.subcore_barrier()

    def body(j, carry):
        drow = dst_v.at[j]
        pltpu.sync_copy(ones_v, deg_sh.at[drow], add=True)
        for k in range(BLK // 16):
            d16 = drow[pl.ds(k * 16, 16)]
            upd_v[pl.ds(k * 16, 16)] = jnp.where(
                d16 == 0, jnp.float32(1.0), jnp.float32(0.0))
        pltpu.sync_copy(upd_v, c0_sh.at[src_v.at[j]], add=True)
        return carry

    lax.fori_loop(0, RPW, body, 0)
    plsc.subcore_barrier()
    pltpu.sync_copy(deg_sh.at[pl.ds(s * NPT, NPT)], out_deg.at[c, s])
    pltpu.sync_copy(c0_sh.at[pl.ds(s * NPT, NPT)], out_c0.at[c, s])


def _sc_gather_scatter_body(srcp, dstp, hs, zer2_h, out_agg,
                            src_v, dst_v, rows0, rows1, agg_sh, sem0, sem1):
    c = lax.axis_index("c")
    s = lax.axis_index("s")
    w = c * NS + s
    pltpu.sync_copy(zer2_h, agg_sh.at[pl.ds(s * NPT, NPT)])
    pltpu.sync_copy(srcp.at[pl.ds(w * RPW, RPW)], src_v)
    pltpu.sync_copy(dstp.at[pl.ds(w * RPW, RPW)], dst_v)
    plsc.subcore_barrier()

    def body(j, carry):
        cp0 = pltpu.async_copy(hs.at[src_v.at[2 * j]], rows0, sem0)
        cp1 = pltpu.async_copy(hs.at[src_v.at[2 * j + 1]], rows1, sem1)
        cp0.wait()
        pltpu.sync_copy(rows0, agg_sh.at[dst_v.at[2 * j]], add=True)
        cp1.wait()
        pltpu.sync_copy(rows1, agg_sh.at[dst_v.at[2 * j + 1]], add=True)
        return carry

    lax.fori_loop(0, RPW // 2, body, 0)
    plsc.subcore_barrier()
    pltpu.sync_copy(agg_sh.at[pl.ds(s * NPT, NPT)], out_agg.at[c, s])


@functools.cache
def _sc_kernels():
    mesh = plsc.VectorSubcoreMesh(core_axis_name="c", subcore_axis_name="s")
    params = pltpu.CompilerParams(use_tc_tiling_on_sc=False)
    deg_c0 = pl.kernel(
        _sc_deg_c0_body,
        mesh=mesh,
        out_type=[
            jax.ShapeDtypeStruct((NC, NS, NPT), jnp.float32),
            jax.ShapeDtypeStruct((NC, NS, NPT), jnp.float32),
        ],
        scratch_types=[
            pltpu.VMEM((RPW, BLK), jnp.int32),
            pltpu.VMEM((RPW, BLK), jnp.int32),
            pltpu.VMEM((BLK,), jnp.float32),
            pltpu.VMEM((BLK,), jnp.float32),
            pltpu.VMEM_SHARED((NPAD,), jnp.float32),
            pltpu.VMEM_SHARED((NPAD,), jnp.float32),
        ],
        compiler_params=params,
    )
    gather_scatter = pl.kernel(
        _sc_gather_scatter_body,
        mesh=mesh,
        out_type=[
            jax.ShapeDtypeStruct((NC, NS, NPT, 32), jnp.float32),
        ],
        scratch_types=[
            pltpu.VMEM((RPW, BLK), jnp.int32),
            pltpu.VMEM((RPW, BLK), jnp.int32),
            pltpu.VMEM((BLK, 32), jnp.float32),
            pltpu.VMEM((BLK, 32), jnp.float32),
            pltpu.VMEM_SHARED((NPAD, 32), jnp.float32),
            pltpu.SemaphoreType.DMA,
            pltpu.SemaphoreType.DMA,
        ],
        compiler_params=params,
    )
    return deg_c0, gather_scatter


def _tc_hs(x_ref, w1_ref, degp_ref, out_ref):
    dp = degp_ref[...]
    deg = dp[0] + dp[1] + jnp.float32(1.0)
    dinv = lax.rsqrt(deg)
    h = jnp.dot(x_ref[...], w1_ref[...], preferred_element_type=jnp.float32)
    out_ref[...] = h * dinv[:N]


def _tc_final(aggp_ref, hs_ref, degp_ref, c0p_ref, b1_ref, w2_ref, b2_ref,
              out_ref):
    dp = degp_ref[...]
    deg = dp[0] + dp[1] + jnp.float32(1.0)
    dinv = lax.rsqrt(deg)
    dv = dinv[:N]
    ap = aggp_ref[...]
    agg = ap[0] + ap[1]
    r1 = jnp.maximum(
        (agg[:N] + hs_ref[...]) * dv + b1_ref[...], jnp.float32(0.0))
    cp = c0p_ref[...]
    c0 = (cp[0] + cp[1])[:N]
    w0 = c0 * dv
    sacc = jnp.sum(r1 * w0, axis=0, keepdims=True)
    d0 = dinv[0:1]
    z = sacc * d0 + r1[0:1] * (d0 * d0)
    out_ref[...] = (
        jnp.dot(z, w2_ref[...], preferred_element_type=jnp.float32)
        + b2_ref[...])


def kernel(x, edge_index, W1, b1, W2, b2):
    src = edge_index[0].astype(jnp.int32)
    dst = edge_index[1].astype(jnp.int32)
    npad_e = EPAD - E
    srcp = jnp.concatenate(
        [src, jnp.zeros((npad_e,), jnp.int32)]).reshape(ROWS, BLK)
    dstp = jnp.concatenate(
        [dst, jnp.full((npad_e,), N, jnp.int32)]).reshape(ROWS, BLK)
    ones_h = jnp.ones((BLK,), jnp.float32)
    zer_h = jnp.zeros((NPT,), jnp.float32)
    zer2_h = jnp.zeros((NPT, 32), jnp.float32)

    deg_c0, gather_scatter = _sc_kernels()
    out_deg, out_c0 = deg_c0(dstp, srcp, ones_h, zer_h)
    degp = out_deg.reshape(NC, NPAD, 1)
    c0p = out_c0.reshape(NC, NPAD, 1)

    hs = pl.pallas_call(
        _tc_hs,
        out_shape=jax.ShapeDtypeStruct((N, 32), jnp.float32),
    )(x, W1, degp)

    (out_agg,) = gather_scatter(srcp, dstp, hs, zer2_h)
    aggp = out_agg.reshape(NC, NPAD, 32)

    out = pl.pallas_call(
        _tc_final,
        out_shape=jax.ShapeDtypeStruct((1, 64), jnp.float32),
    )(aggp, hs, degp, c0p, b1.reshape(1, 32), W2, b2.reshape(1, 64))
    return out.reshape(64)


# speedup vs baseline: 68.5622x; 1.9094x over previous
import functools

import jax
import jax.numpy as jnp
from jax import lax
from jax.experimental import pallas as pl
from jax.experimental.pallas import tpu as pltpu
from jax.experimental.pallas import tpu_sc as plsc

N = 10000
NPAD = 10240
E = 320000
NC, NS = 2, 16
NW = NC * NS
NPT = NPAD // NS

BLKA = 80
RA = E // (NW * BLKA)
BLKB = 125
RB = E // (NW * BLKB)
NBUF = 4


def _sc_deg_c0_body(dstp, srcp, ones_h, zer_h, out_deg, out_c0,
                    dst_v, src_v, ones_v, upd_v, deg_sh, c0_sh):
    c = lax.axis_index("c")
    s = lax.axis_index("s")
    w = c * NS + s
    pltpu.sync_copy(zer_h, deg_sh.at[pl.ds(s * NPT, NPT)])
    pltpu.sync_copy(zer_h, c0_sh.at[pl.ds(s * NPT, NPT)])
    pltpu.sync_copy(ones_h, ones_v)
    pltpu.sync_copy(dstp.at[pl.ds(w * RA, RA)], dst_v)
    pltpu.sync_copy(srcp.at[pl.ds(w * RA, RA)], src_v)
    plsc.subcore_barrier()

    def body(j, carry):
        drow = dst_v.at[j]
        pltpu.sync_copy(ones_v, deg_sh.at[drow], add=True)
        cnt = jnp.int32(0)
        for k in range(BLKA // 16):
            d16 = drow[pl.ds(k * 16, 16)]
            hit = d16 == 0
            upd_v[pl.ds(k * 16, 16)] = jnp.where(
                hit, jnp.float32(1.0), jnp.float32(0.0))
            cnt = cnt + jnp.sum(jnp.where(hit, jnp.int32(1), jnp.int32(0)))

        @pl.when(cnt > 0)
        def _():
            pltpu.sync_copy(upd_v, c0_sh.at[src_v.at[j]], add=True)

        return carry

    lax.fori_loop(0, RA, body, 0)
    plsc.subcore_barrier()
    pltpu.sync_copy(deg_sh.at[pl.ds(s * NPT, NPT)], out_deg.at[c, s])
    pltpu.sync_copy(c0_sh.at[pl.ds(s * NPT, NPT)], out_c0.at[c, s])


def _sc_gather_scatter_body(srcp, dstp, hs, zer2_h, out_agg,
                            src_v, dst_v, rows, agg_sh, sems):
    c = lax.axis_index("c")
    s = lax.axis_index("s")
    w = c * NS + s
    pltpu.sync_copy(zer2_h, agg_sh.at[pl.ds(s * NPT, NPT)])
    pltpu.sync_copy(srcp.at[pl.ds(w * RB, RB)], src_v)
    pltpu.sync_copy(dstp.at[pl.ds(w * RB, RB)], dst_v)
    plsc.subcore_barrier()

    for b in range(NBUF):
        pltpu.async_copy(hs.at[src_v.at[b]], rows[b], sems[b])

    def body(g, carry):
        for b in range(NBUF):
            j = NBUF * g + b
            pltpu.make_async_copy(hs.at[src_v.at[j]], rows[b], sems[b]).wait()
            pltpu.sync_copy(rows[b], agg_sh.at[dst_v.at[j]], add=True)

            @pl.when(g < RB // NBUF - 1)
            def _():
                pltpu.async_copy(
                    hs.at[src_v.at[j + NBUF]], rows[b], sems[b])

        return carry

    lax.fori_loop(0, RB // NBUF, body, 0)
    plsc.subcore_barrier()
    pltpu.sync_copy(agg_sh.at[pl.ds(s * NPT, NPT)], out_agg.at[c, s])


@functools.cache
def _sc_kernels():
    mesh = plsc.VectorSubcoreMesh(core_axis_name="c", subcore_axis_name="s")
    params = pltpu.CompilerParams(
        use_tc_tiling_on_sc=False, needs_layout_passes=False)
    deg_c0 = pl.kernel(
        _sc_deg_c0_body,
        mesh=mesh,
        out_type=[
            jax.ShapeDtypeStruct((NC, NS, NPT), jnp.float32),
            jax.ShapeDtypeStruct((NC, NS, NPT), jnp.float32),
        ],
        scratch_types=[
            pltpu.VMEM((RA, BLKA), jnp.int32),
            pltpu.VMEM((RA, BLKA), jnp.int32),
            pltpu.VMEM((BLKA,), jnp.float32),
            pltpu.VMEM((BLKA,), jnp.float32),
            pltpu.VMEM_SHARED((NPAD,), jnp.float32),
            pltpu.VMEM_SHARED((NPAD,), jnp.float32),
        ],
        compiler_params=params,
    )
    gather_scatter = pl.kernel(
        _sc_gather_scatter_body,
        mesh=mesh,
        out_type=[
            jax.ShapeDtypeStruct((NC, NS, NPT, 32), jnp.float32),
        ],
        scratch_types=[
            pltpu.VMEM((RB, BLKB), jnp.int32),
            pltpu.VMEM((RB, BLKB), jnp.int32),
            [pltpu.VMEM((BLKB, 32), jnp.float32) for _ in range(NBUF)],
            pltpu.VMEM_SHARED((NPAD, 32), jnp.float32),
            [pltpu.SemaphoreType.DMA for _ in range(NBUF)],
        ],
        compiler_params=params,
    )
    return deg_c0, gather_scatter


def _tc_h(x_ref, w1_ref, out_ref):
    out_ref[...] = jnp.dot(
        x_ref[...], w1_ref[...], preferred_element_type=jnp.float32)


def _tc_hs(h_ref, degp_ref, out_ref):
    dp = degp_ref[...]
    deg = dp[0] + dp[1] + jnp.float32(1.0)
    dinv = lax.rsqrt(deg)
    out_ref[...] = h_ref[...] * dinv[:N]


def _tc_final(aggp_ref, hs_ref, degp_ref, c0p_ref, b1_ref, w2_ref, b2_ref,
              out_ref):
    dp = degp_ref[...]
    deg = dp[0] + dp[1] + jnp.float32(1.0)
    dinv = lax.rsqrt(deg)
    dv = dinv[:N]
    ap = aggp_ref[...]
    agg = ap[0] + ap[1]
    r1 = jnp.maximum(
        (agg[:N] + hs_ref[...]) * dv + b1_ref[...], jnp.float32(0.0))
    cp = c0p_ref[...]
    c0 = (cp[0] + cp[1])[:N]
    w0 = c0 * dv
    sacc = jnp.sum(r1 * w0, axis=0, keepdims=True)
    d0 = dinv[0:1]
    z = sacc * d0 + r1[0:1] * (d0 * d0)
    out_ref[...] = (
        jnp.dot(z, w2_ref[...], preferred_element_type=jnp.float32)
        + b2_ref[...])


def kernel(x, edge_index, W1, b1, W2, b2):
    src = edge_index[0].astype(jnp.int32)
    dst = edge_index[1].astype(jnp.int32)
    srcp_a = src.reshape(NW * RA, BLKA)
    dstp_a = dst.reshape(NW * RA, BLKA)
    srcp_b = src.reshape(NW * RB, BLKB)
    dstp_b = dst.reshape(NW * RB, BLKB)
    ones_h = jnp.ones((BLKA,), jnp.float32)
    zer_h = jnp.zeros((NPT,), jnp.float32)
    zer2_h = jnp.zeros((NPT, 32), jnp.float32)

    deg_c0, gather_scatter = _sc_kernels()
    out_deg, out_c0 = deg_c0(dstp_a, srcp_a, ones_h, zer_h)
    degp = out_deg.reshape(NC, NPAD, 1)
    c0p = out_c0.reshape(NC, NPAD, 1)

    h = pl.pallas_call(
        _tc_h,
        out_shape=jax.ShapeDtypeStruct((N, 32), jnp.float32),
    )(x, W1)
    hs = pl.pallas_call(
        _tc_hs,
        out_shape=jax.ShapeDtypeStruct((N, 32), jnp.float32),
    )(h, degp)

    (out_agg,) = gather_scatter(srcp_b, dstp_b, hs, zer2_h)
    aggp = out_agg.reshape(NC, NPAD, 32)

    out = pl.pallas_call(
        _tc_final,
        out_shape=jax.ShapeDtypeStruct((1, 64), jnp.float32),
    )(aggp, hs, degp, c0p, b1.reshape(1, 32), W2, b2.reshape(1, 64))
    return out.reshape(64)


# speedup vs baseline: 69.4647x; 1.0132x over previous
import functools

import jax
import jax.numpy as jnp
from jax import lax
from jax.experimental import pallas as pl
from jax.experimental.pallas import tpu as pltpu
from jax.experimental.pallas import tpu_sc as plsc

N = 10000
NPAD = 10240
E = 320000
NC, NS = 2, 16
NW = NC * NS
NPT = NPAD // NS

BLKA = 80
RA = E // (NW * BLKA)
BLKB = 125
RB = E // (NW * BLKB)
NBUF = 8


def _sc_deg_c0_body(dstp, srcp, ones_h, zer_h, out_deg, out_c0,
                    dst_v, src_v, ones_v, upd_v, deg_sh, c0_sh):
    c = lax.axis_index("c")
    s = lax.axis_index("s")
    w = c * NS + s
    pltpu.sync_copy(zer_h, deg_sh.at[pl.ds(s * NPT, NPT)])
    pltpu.sync_copy(zer_h, c0_sh.at[pl.ds(s * NPT, NPT)])
    pltpu.sync_copy(ones_h, ones_v)
    pltpu.sync_copy(dstp.at[pl.ds(w * RA, RA)], dst_v)
    pltpu.sync_copy(srcp.at[pl.ds(w * RA, RA)], src_v)
    plsc.subcore_barrier()

    def body(j, carry):
        drow = dst_v.at[j]
        pltpu.sync_copy(ones_v, deg_sh.at[drow], add=True)
        cnt = jnp.int32(0)
        for k in range(BLKA // 16):
            d16 = drow[pl.ds(k * 16, 16)]
            hit = d16 == 0
            upd_v[pl.ds(k * 16, 16)] = jnp.where(
                hit, jnp.float32(1.0), jnp.float32(0.0))
            cnt = cnt + jnp.sum(jnp.where(hit, jnp.int32(1), jnp.int32(0)))

        @pl.when(cnt > 0)
        def _():
            pltpu.sync_copy(upd_v, c0_sh.at[src_v.at[j]], add=True)

        return carry

    lax.fori_loop(0, RA, body, 0)
    plsc.subcore_barrier()
    pltpu.sync_copy(deg_sh.at[pl.ds(s * NPT, NPT)], out_deg.at[c, s])
    pltpu.sync_copy(c0_sh.at[pl.ds(s * NPT, NPT)], out_c0.at[c, s])


def _sc_gather_scatter_body(srcp, dstp, hs, zer2_h, out_agg,
                            src_v, dst_v, rows, agg_sh, sems):
    c = lax.axis_index("c")
    s = lax.axis_index("s")
    w = c * NS + s
    pltpu.sync_copy(zer2_h, agg_sh.at[pl.ds(s * NPT, NPT)])
    pltpu.sync_copy(srcp.at[pl.ds(w * RB, RB)], src_v)
    pltpu.sync_copy(dstp.at[pl.ds(w * RB, RB)], dst_v)
    plsc.subcore_barrier()

    for b in range(NBUF):
        pltpu.async_copy(hs.at[src_v.at[b]], rows[b], sems[b])

    def body(g, carry):
        for b in range(NBUF):
            j = NBUF * g + b
            pltpu.make_async_copy(hs.at[src_v.at[j]], rows[b], sems[b]).wait()
            pltpu.sync_copy(rows[b], agg_sh.at[dst_v.at[j]], add=True)

            @pl.when(g < RB // NBUF - 1)
            def _():
                pltpu.async_copy(
                    hs.at[src_v.at[j + NBUF]], rows[b], sems[b])

        return carry

    lax.fori_loop(0, RB // NBUF, body, 0)
    plsc.subcore_barrier()
    pltpu.sync_copy(agg_sh.at[pl.ds(s * NPT, NPT)], out_agg.at[c, s])


@functools.cache
def _sc_kernels():
    mesh = plsc.VectorSubcoreMesh(core_axis_name="c", subcore_axis_name="s")
    params = pltpu.CompilerParams(
        use_tc_tiling_on_sc=False, needs_layout_passes=False,
        skip_device_barrier=True)
    deg_c0 = pl.kernel(
        _sc_deg_c0_body,
        mesh=mesh,
        out_type=[
            jax.ShapeDtypeStruct((NC, NS, NPT), jnp.float32),
            jax.ShapeDtypeStruct((NC, NS, NPT), jnp.float32),
        ],
        scratch_types=[
            pltpu.VMEM((RA, BLKA), jnp.int32),
            pltpu.VMEM((RA, BLKA), jnp.int32),
            pltpu.VMEM((BLKA,), jnp.float32),
            pltpu.VMEM((BLKA,), jnp.float32),
            pltpu.VMEM_SHARED((NPAD,), jnp.float32),
            pltpu.VMEM_SHARED((NPAD,), jnp.float32),
        ],
        compiler_params=params,
    )
    gather_scatter = pl.kernel(
        _sc_gather_scatter_body,
        mesh=mesh,
        out_type=[
            jax.ShapeDtypeStruct((NC, NS, NPT, 32), jnp.float32),
        ],
        scratch_types=[
            pltpu.VMEM((RB, BLKB), jnp.int32),
            pltpu.VMEM((RB, BLKB), jnp.int32),
            [pltpu.VMEM((BLKB, 32), jnp.float32) for _ in range(NBUF)],
            pltpu.VMEM_SHARED((NPAD, 32), jnp.float32),
            [pltpu.SemaphoreType.DMA for _ in range(NBUF)],
        ],
        compiler_params=params,
    )
    return deg_c0, gather_scatter


def _tc_hs(x_ref, w1_ref, degp_ref, out_ref):
    dp = degp_ref[...]
    deg = dp[0] + dp[1] + jnp.float32(1.0)
    dinv = lax.rsqrt(deg)
    h = jnp.dot(x_ref[...], w1_ref[...], preferred_element_type=jnp.float32)
    out_ref[...] = h * dinv[:N]


def _tc_final(aggp_ref, hs_ref, degp_ref, c0p_ref, b1_ref, w2_ref, b2_ref,
              out_ref):
    dp = degp_ref[...]
    deg = dp[0] + dp[1] + jnp.float32(1.0)
    dinv = lax.rsqrt(deg)
    dv = dinv[:N]
    ap = aggp_ref[...]
    agg = ap[0] + ap[1]
    r1 = jnp.maximum(
        (agg[:N] + hs_ref[...]) * dv + b1_ref[...], jnp.float32(0.0))
    cp = c0p_ref[...]
    c0 = (cp[0] + cp[1])[:N]
    w0 = c0 * dv
    sacc = jnp.sum(r1 * w0, axis=0, keepdims=True)
    d0 = dinv[0:1]
    z = sacc * d0 + r1[0:1] * (d0 * d0)
    out_ref[...] = (
        jnp.dot(z, w2_ref[...], preferred_element_type=jnp.float32)
        + b2_ref[...])


def kernel(x, edge_index, W1, b1, W2, b2):
    src = edge_index[0].astype(jnp.int32)
    dst = edge_index[1].astype(jnp.int32)
    srcp_a = src.reshape(NW * RA, BLKA)
    dstp_a = dst.reshape(NW * RA, BLKA)
    srcp_b = src.reshape(NW * RB, BLKB)
    dstp_b = dst.reshape(NW * RB, BLKB)
    ones_h = jnp.ones((BLKA,), jnp.float32)
    zer_h = jnp.zeros((NPT,), jnp.float32)
    zer2_h = jnp.zeros((NPT, 32), jnp.float32)

    deg_c0, gather_scatter = _sc_kernels()
    out_deg, out_c0 = deg_c0(dstp_a, srcp_a, ones_h, zer_h)
    degp = out_deg.reshape(NC, NPAD, 1)
    c0p = out_c0.reshape(NC, NPAD, 1)

    hs = pl.pallas_call(
        _tc_hs,
        out_shape=jax.ShapeDtypeStruct((N, 32), jnp.float32),
    )(x, W1, degp)

    (out_agg,) = gather_scatter(srcp_b, dstp_b, hs, zer2_h)
    aggp = out_agg.reshape(NC, NPAD, 32)

    out = pl.pallas_call(
        _tc_final,
        out_shape=jax.ShapeDtypeStruct((1, 64), jnp.float32),
    )(aggp, hs, degp, c0p, b1.reshape(1, 32), W2, b2.reshape(1, 64))
    return out.reshape(64)


# speedup vs baseline: 72.4222x; 1.0426x over previous
import functools

import jax
import jax.numpy as jnp
from jax import lax
from jax.experimental import pallas as pl
from jax.experimental.pallas import tpu as pltpu
from jax.experimental.pallas import tpu_sc as plsc

N = 10000
NPAD = 10240
E = 320000
NC, NS = 2, 16
NW = NC * NS
NPT = NPAD // NS
EPW = E // NW
BLK = 80
NBLK = EPW // BLK
NBUF = 5


def _sc_deg_c0_body(ei, ones_h, zer_h, out_deg, out_c0,
                    dst_v, src_v, ones_v, upd_v, deg_sh, c0_sh):
    c = lax.axis_index("c")
    s = lax.axis_index("s")
    w = c * NS + s
    pltpu.sync_copy(zer_h, deg_sh.at[pl.ds(s * NPT, NPT)])
    pltpu.sync_copy(zer_h, c0_sh.at[pl.ds(s * NPT, NPT)])
    pltpu.sync_copy(ones_h, ones_v)
    pltpu.sync_copy(ei.at[1, pl.ds(w * EPW, EPW)], dst_v)
    pltpu.sync_copy(ei.at[0, pl.ds(w * EPW, EPW)], src_v)
    plsc.subcore_barrier()

    def body(j, carry):
        base = j * BLK
        pltpu.sync_copy(ones_v, deg_sh.at[dst_v.at[pl.ds(base, BLK)]],
                        add=True)
        cnt = jnp.int32(0)
        for k in range(BLK // 16):
            d16 = dst_v[pl.ds(base + k * 16, 16)]
            hit = d16 == 0
            upd_v[pl.ds(k * 16, 16)] = jnp.where(
                hit, jnp.float32(1.0), jnp.float32(0.0))
            cnt = cnt + jnp.sum(jnp.where(hit, jnp.int32(1), jnp.int32(0)))

        @pl.when(cnt > 0)
        def _():
            pltpu.sync_copy(upd_v, c0_sh.at[src_v.at[pl.ds(base, BLK)]],
                            add=True)

        return carry

    lax.fori_loop(0, NBLK, body, 0)
    plsc.subcore_barrier()
    pltpu.sync_copy(deg_sh.at[pl.ds(s * NPT, NPT)],
                    out_deg.at[c, pl.ds(s * NPT, NPT)])
    pltpu.sync_copy(c0_sh.at[pl.ds(s * NPT, NPT)],
                    out_c0.at[c, pl.ds(s * NPT, NPT)])


def _sc_gather_scatter_body(ei, hs, zer2_h, out_agg,
                            src_v, dst_v, rows, agg_sh, sems):
    c = lax.axis_index("c")
    s = lax.axis_index("s")
    w = c * NS + s
    pltpu.sync_copy(zer2_h, agg_sh.at[pl.ds(s * NPT, NPT)])
    pltpu.sync_copy(ei.at[0, pl.ds(w * EPW, EPW)], src_v)
    pltpu.sync_copy(ei.at[1, pl.ds(w * EPW, EPW)], dst_v)
    plsc.subcore_barrier()

    for b in range(NBUF):
        pltpu.async_copy(hs.at[src_v.at[pl.ds(b * BLK, BLK)]], rows[b],
                         sems[b])

    def body(g, carry):
        for b in range(NBUF):
            j = NBUF * g + b
            pltpu.make_async_copy(
                hs.at[src_v.at[pl.ds(j * BLK, BLK)]], rows[b],
                sems[b]).wait()
            pltpu.sync_copy(rows[b], agg_sh.at[dst_v.at[pl.ds(j * BLK, BLK)]],
                            add=True)

            @pl.when(g < NBLK // NBUF - 1)
            def _():
                pltpu.async_copy(
                    hs.at[src_v.at[pl.ds((j + NBUF) * BLK, BLK)]], rows[b],
                    sems[b])

        return carry

    lax.fori_loop(0, NBLK // NBUF, body, 0)
    plsc.subcore_barrier()
    pltpu.sync_copy(agg_sh.at[pl.ds(s * NPT, NPT)],
                    out_agg.at[c, pl.ds(s * NPT, NPT)])


@functools.cache
def _sc_kernels():
    mesh = plsc.VectorSubcoreMesh(core_axis_name="c", subcore_axis_name="s")
    params = pltpu.CompilerParams(
        use_tc_tiling_on_sc=False, needs_layout_passes=False,
        skip_device_barrier=True)
    deg_c0 = pl.kernel(
        _sc_deg_c0_body,
        mesh=mesh,
        out_type=[
            jax.ShapeDtypeStruct((NC, NPAD), jnp.float32),
            jax.ShapeDtypeStruct((NC, NPAD), jnp.float32),
        ],
        scratch_types=[
            pltpu.VMEM((EPW,), jnp.int32),
            pltpu.VMEM((EPW,), jnp.int32),
            pltpu.VMEM((BLK,), jnp.float32),
            pltpu.VMEM((BLK,), jnp.float32),
            pltpu.VMEM_SHARED((NPAD,), jnp.float32),
            pltpu.VMEM_SHARED((NPAD,), jnp.float32),
        ],
        compiler_params=params,
    )
    gather_scatter = pl.kernel(
        _sc_gather_scatter_body,
        mesh=mesh,
        out_type=[
            jax.ShapeDtypeStruct((NC, NPAD, 32), jnp.float32),
        ],
        scratch_types=[
            pltpu.VMEM((EPW,), jnp.int32),
            pltpu.VMEM((EPW,), jnp.int32),
            [pltpu.VMEM((BLK, 32), jnp.float32) for _ in range(NBUF)],
            pltpu.VMEM_SHARED((NPAD, 32), jnp.float32),
            [pltpu.SemaphoreType.DMA for _ in range(NBUF)],
        ],
        compiler_params=params,
    )
    return deg_c0, gather_scatter


def _tc_h(x_ref, w1_ref, out_ref):
    out_ref[...] = jnp.dot(
        x_ref[...], w1_ref[...], preferred_element_type=jnp.float32)


def _tc_hs(h_ref, degp_ref, out_ref):
    dp = degp_ref[...]
    deg = dp[0] + dp[1] + jnp.float32(1.0)
    dinv = lax.rsqrt(deg)
    out_ref[...] = h_ref[...] * dinv[:N]


def _tc_final(aggp_ref, hs_ref, degp_ref, c0p_ref, b1_ref, w2_ref, b2_ref,
              out_ref):
    dp = degp_ref[...]
    deg = dp[0] + dp[1] + jnp.float32(1.0)
    dinv = lax.rsqrt(deg)
    dv = dinv[:N]
    ap = aggp_ref[...]
    agg = ap[0] + ap[1]
    r1 = jnp.maximum(
        (agg[:N] + hs_ref[...]) * dv + b1_ref[...], jnp.float32(0.0))
    cp = c0p_ref[...]
    c0 = (cp[0] + cp[1])[:N]
    w0 = c0 * dv
    sacc = jnp.sum(r1 * w0, axis=0, keepdims=True)
    d0 = dinv[0:1]
    z = sacc * d0 + r1[0:1] * (d0 * d0)
    out_ref[...] = (
        jnp.dot(z, w2_ref[...], preferred_element_type=jnp.float32)
        + b2_ref[...])


def kernel(x, edge_index, W1, b1, W2, b2):
    ei = edge_index.astype(jnp.int32)
    ones_h = jnp.ones((BLK,), jnp.float32)
    zer_h = jnp.zeros((NPT,), jnp.float32)
    zer2_h = jnp.zeros((NPT, 32), jnp.float32)

    deg_c0, gather_scatter = _sc_kernels()
    out_deg, out_c0 = deg_c0(ei, ones_h, zer_h)
    degp = out_deg.reshape(NC, NPAD, 1)
    c0p = out_c0.reshape(NC, NPAD, 1)

    h = pl.pallas_call(
        _tc_h,
        out_shape=jax.ShapeDtypeStruct((N, 32), jnp.float32),
    )(x, W1)
    hs = pl.pallas_call(
        _tc_hs,
        out_shape=jax.ShapeDtypeStruct((N, 32), jnp.float32),
    )(h, degp)

    (out_agg,) = gather_scatter(ei, hs, zer2_h)

    out = pl.pallas_call(
        _tc_final,
        out_shape=jax.ShapeDtypeStruct((1, 64), jnp.float32),
    )(out_agg, hs, degp, c0p, b1.reshape(1, 32), W2, b2.reshape(1, 64))
    return out.reshape(64)


# speedup vs baseline: 86.6432x; 1.1964x over previous
import functools

import jax
import jax.numpy as jnp
from jax import lax
from jax.experimental import pallas as pl
from jax.experimental.pallas import tpu as pltpu
from jax.experimental.pallas import tpu_sc as plsc

N = 10000
NPAD = 10240
E = 320000
NC, NS = 2, 16
NW = NC * NS
NPT = NPAD // NS
EPW = E // NW
BLK = 80
NBLK = EPW // BLK
NBUF = 5


def _sc_deg_c0_body(ei, ones_h, zer_h, out_deg, out_c0,
                    dst_v, src_v, ones_v, upd_v, deg_sh, c0_sh):
    c = lax.axis_index("c")
    s = lax.axis_index("s")
    w = c * NS + s
    pltpu.sync_copy(zer_h, deg_sh.at[pl.ds(s * NPT, NPT)])
    pltpu.sync_copy(zer_h, c0_sh.at[pl.ds(s * NPT, NPT)])
    pltpu.sync_copy(ones_h, ones_v)
    pltpu.sync_copy(ei.at[1, pl.ds(w * EPW, EPW)], dst_v)
    pltpu.sync_copy(ei.at[0, pl.ds(w * EPW, EPW)], src_v)
    plsc.subcore_barrier()

    def body(j, carry):
        base = j * BLK
        pltpu.sync_copy(ones_v, deg_sh.at[dst_v.at[pl.ds(base, BLK)]],
                        add=True)
        cnt = jnp.int32(0)
        for k in range(BLK // 16):
            d16 = dst_v[pl.ds(base + k * 16, 16)]
            hit = d16 == 0
            upd_v[pl.ds(k * 16, 16)] = jnp.where(
                hit, jnp.float32(1.0), jnp.float32(0.0))
            cnt = cnt + jnp.sum(jnp.where(hit, jnp.int32(1), jnp.int32(0)))

        @pl.when(cnt > 0)
        def _():
            pltpu.sync_copy(upd_v, c0_sh.at[src_v.at[pl.ds(base, BLK)]],
                            add=True)

        return carry

    lax.fori_loop(0, NBLK, body, 0)
    plsc.subcore_barrier()
    pltpu.sync_copy(deg_sh.at[pl.ds(s * NPT, NPT)],
                    out_deg.at[c, pl.ds(s * NPT, NPT)])
    pltpu.sync_copy(c0_sh.at[pl.ds(s * NPT, NPT)],
                    out_c0.at[c, pl.ds(s * NPT, NPT)])


def _sc_gather_scatter_body(ei, hs, zer2_h, out_agg,
                            src_v, dst_v, rows, agg_sh, sems):
    c = lax.axis_index("c")
    s = lax.axis_index("s")
    w = c * NS + s
    pltpu.sync_copy(zer2_h, agg_sh.at[pl.ds(s * NPT, NPT)])
    pltpu.sync_copy(ei.at[0, pl.ds(w * EPW, EPW)], src_v)
    pltpu.sync_copy(ei.at[1, pl.ds(w * EPW, EPW)], dst_v)
    plsc.subcore_barrier()

    for b in range(NBUF):
        pltpu.async_copy(hs.at[src_v.at[pl.ds(b * BLK, BLK)]], rows[b],
                         sems[b])

    def body(g, carry):
        for b in range(NBUF):
            j = NBUF * g + b
            pltpu.make_async_copy(
                hs.at[src_v.at[pl.ds(j * BLK, BLK)]], rows[b],
                sems[b]).wait()
            pltpu.sync_copy(rows[b], agg_sh.at[dst_v.at[pl.ds(j * BLK, BLK)]],
                            add=True)

            @pl.when(g < NBLK // NBUF - 1)
            def _():
                pltpu.async_copy(
                    hs.at[src_v.at[pl.ds((j + NBUF) * BLK, BLK)]], rows[b],
                    sems[b])

        return carry

    lax.fori_loop(0, NBLK // NBUF, body, 0)
    plsc.subcore_barrier()
    pltpu.sync_copy(agg_sh.at[pl.ds(s * NPT, NPT)],
                    out_agg.at[c, pl.ds(s * NPT, NPT)])


@functools.cache
def _sc_kernels():
    mesh = plsc.VectorSubcoreMesh(core_axis_name="c", subcore_axis_name="s")
    params = pltpu.CompilerParams(
        use_tc_tiling_on_sc=False, needs_layout_passes=False,
        skip_device_barrier=True)
    deg_c0 = pl.kernel(
        _sc_deg_c0_body,
        mesh=mesh,
        out_type=[
            jax.ShapeDtypeStruct((NC, NPAD), jnp.float32),
            jax.ShapeDtypeStruct((NC, NPAD), jnp.float32),
        ],
        scratch_types=[
            pltpu.VMEM((EPW,), jnp.int32),
            pltpu.VMEM((EPW,), jnp.int32),
            pltpu.VMEM((BLK,), jnp.float32),
            pltpu.VMEM((BLK,), jnp.float32),
            pltpu.VMEM_SHARED((NPAD,), jnp.float32),
            pltpu.VMEM_SHARED((NPAD,), jnp.float32),
        ],
        compiler_params=params,
    )
    gather_scatter = pl.kernel(
        _sc_gather_scatter_body,
        mesh=mesh,
        out_type=[
            jax.ShapeDtypeStruct((NC, NPAD, 32), jnp.float32),
        ],
        scratch_types=[
            pltpu.VMEM((EPW,), jnp.int32),
            pltpu.VMEM((EPW,), jnp.int32),
            [pltpu.VMEM((BLK, 32), jnp.float32) for _ in range(NBUF)],
            pltpu.VMEM_SHARED((NPAD, 32), jnp.float32),
            [pltpu.SemaphoreType.DMA for _ in range(NBUF)],
        ],
        compiler_params=params,
    )
    return deg_c0, gather_scatter


def _tc_h(x_ref, w1_ref, out_ref):
    out_ref[...] = jnp.dot(
        x_ref[...], w1_ref[...], preferred_element_type=jnp.float32)


def _tc_hs(h_ref, degp_ref, out_ref):
    dp = degp_ref[...]
    deg = dp[0] + dp[1] + jnp.float32(1.0)
    dinv = lax.rsqrt(deg)
    out_ref[...] = h_ref[...] * dinv[:N].reshape(N, 1)


def _tc_final(aggp_ref, hs_ref, degp_ref, c0p_ref, b1_ref, w2_ref, b2_ref,
              out_ref):
    dp = degp_ref[...]
    deg = dp[0] + dp[1] + jnp.float32(1.0)
    dinv = lax.rsqrt(deg)
    dv = dinv[:N].reshape(N, 1)
    ap = aggp_ref[...]
    agg = ap[0] + ap[1]
    r1 = jnp.maximum(
        (agg[:N] + hs_ref[...]) * dv + b1_ref[...], jnp.float32(0.0))
    cp = c0p_ref[...]
    c0 = (cp[0] + cp[1])[:N].reshape(N, 1)
    w0 = c0 * dv
    sacc = jnp.sum(r1 * w0, axis=0, keepdims=True)
    d0 = dinv[0:1].reshape(1, 1)
    z = sacc * d0 + r1[0:1] * (d0 * d0)
    out_ref[...] = (
        jnp.dot(z, w2_ref[...], preferred_element_type=jnp.float32)
        + b2_ref[...])


def kernel(x, edge_index, W1, b1, W2, b2):
    ei = edge_index.astype(jnp.int32)
    ones_h = jnp.ones((BLK,), jnp.float32)
    zer_h = jnp.zeros((NPT,), jnp.float32)
    zer2_h = jnp.zeros((NPT, 32), jnp.float32)

    deg_c0, gather_scatter = _sc_kernels()
    degp, c0p = deg_c0(ei, ones_h, zer_h)

    h = pl.pallas_call(
        _tc_h,
        out_shape=jax.ShapeDtypeStruct((N, 32), jnp.float32),
    )(x, W1)
    hs = pl.pallas_call(
        _tc_hs,
        out_shape=jax.ShapeDtypeStruct((N, 32), jnp.float32),
    )(h, degp)

    (out_agg,) = gather_scatter(ei, hs, zer2_h)

    out = pl.pallas_call(
        _tc_final,
        out_shape=jax.ShapeDtypeStruct((1, 64), jnp.float32),
    )(out_agg, hs, degp, c0p, b1.reshape(1, 32), W2, b2.reshape(1, 64))
    return out.reshape(64)
